# Initial kernel scaffold; baseline (speedup 1.0000x reference)
import jax
import jax.numpy as jnp
from jax.experimental import pallas as pl


def kernel(node_feats, node_types, edge_index, edge_vals, idxes_seq, idxes_res, W_types, b_types, affine_W, affine_b, ln_scale, ln_bias, attn_W1, attn_b1, attn_W2, attn_b2, cls_W, cls_b):
    raise NotImplementedError("write your pallas kernel here")


# speedup vs baseline: 3.1463x; 3.1463x over previous
import functools

import jax
import jax.numpy as jnp
from jax import lax
from jax.experimental import pallas as pl
from jax.experimental.pallas import tpu as pltpu
from jax.experimental.pallas import tpu_sc as plsc

N = 10000
E = 320000
IN_DIM = 128
H = 128
T = 4
C = 16
ATT = 64
N_META = 2

BN = 1000
K = 80


def _make_sc_spmm(nout, ntables, terms, nc, ns):
  nw = nc * ns
  we = E // nw
  nch = we // K
  rpw = (N // ns) & ~7
  rem = N - rpw * ns
  mesh = plsc.VectorSubcoreMesh(core_axis_name="c", subcore_axis_name="s")

  @functools.partial(
      pl.kernel,
      mesh=mesh,
      out_type=jax.ShapeDtypeStruct((nout, nc, N, H), jnp.float32),
      scratch_types=[
          pltpu.VMEM((K,), jnp.int32),
          pltpu.VMEM((K,), jnp.int32),
          pltpu.VMEM((K,), jnp.float32),
          pltpu.VMEM((K, H), jnp.float32),
          pltpu.VMEM_SHARED((N, H), jnp.float32),
          pltpu.SemaphoreType.DMA,
      ],
  )
  def sc_fn(*refs):
    tables = refs[:ntables]
    src_h, dst_h, val_h, zeros_h, out_h = refs[ntables:ntables + 5]
    src_v, dst_v, val_v, rows_v, acc_s, sem = refs[ntables + 5:]
    cid = lax.axis_index("c")
    sid = lax.axis_index("s")
    wid = sid * nc + cid

    for o in range(nout):
      pltpu.sync_copy(zeros_h.at[pl.ds(sid * rpw, rpw)],
                      acc_s.at[pl.ds(sid * rpw, rpw)])
      if rem:
        @pl.when(sid == ns - 1)
        def _():
          pltpu.sync_copy(zeros_h.at[pl.ds(N - rem, rem)],
                          acc_s.at[pl.ds(N - rem, rem)])
      plsc.subcore_barrier()
      for t, (oo, tbl) in enumerate(terms):
        if oo != o:
          continue
        table = tables[tbl]
        base = t * E + wid * we

        def chunk(ci, _, base=base, table=table):
          off = base + ci * K
          pltpu.sync_copy(src_h.at[pl.ds(off, K)], src_v)
          pltpu.sync_copy(dst_h.at[pl.ds(off, K)], dst_v)
          pltpu.sync_copy(val_h.at[pl.ds(off, K)], val_v)
          pltpu.async_copy(table.at[src_v], rows_v, sem).wait()

          for i16 in range(K // 16):
            vv = val_v[pl.ds(i16 * 16, 16)]
            for j in range(16):
              v = vv[j]
              i = i16 * 16 + j
              for g in range(H // 16):
                sl = pl.ds(g * 16, 16)
                rows_v[i, sl] = rows_v[i, sl] * v
          pltpu.sync_copy(rows_v, acc_s.at[dst_v], add=True)
          return 0

        lax.fori_loop(0, nch, chunk, 0)
      plsc.subcore_barrier()
      pltpu.sync_copy(acc_s.at[pl.ds(sid * rpw, rpw)],
                      out_h.at[o, cid, pl.ds(sid * rpw, rpw)])
      if rem:
        @pl.when(sid == ns - 1)
        def _():
          pltpu.sync_copy(acc_s.at[pl.ds(N - rem, rem)],
                          out_h.at[o, cid, pl.ds(N - rem, rem)])
      plsc.subcore_barrier()

  return sc_fn


def _tc_a_body(nf_ref, nt_ref, wt_ref, bt_ref, aw_ref, ab_ref,
               x0_ref, x1_ref):
  x = nf_ref[...]
  t = nt_ref[...]
  h = jnp.zeros((BN, H), jnp.float32)
  for tt in range(T):
    ht = jnp.dot(x, wt_ref[tt], preferred_element_type=jnp.float32)
    ht = ht + bt_ref[tt]
    h = jnp.where(t == tt, ht, h)
  x0_ref[...] = jnp.dot(h, aw_ref[0],
                        preferred_element_type=jnp.float32) + ab_ref[0]
  x1_ref[...] = jnp.dot(h, aw_ref[1],
                        preferred_element_type=jnp.float32) + ab_ref[1]


def _tc_a(node_feats, node_types, w_types, b_types, affine_w, affine_b):
  grid = (N // BN,)
  return pl.pallas_call(
      _tc_a_body,
      grid=grid,
      in_specs=[
          pl.BlockSpec((BN, IN_DIM), lambda i: (i, 0)),
          pl.BlockSpec((BN, 1), lambda i: (i, 0)),
          pl.BlockSpec((T, IN_DIM, H), lambda i: (0, 0, 0)),
          pl.BlockSpec((T, 1, H), lambda i: (0, 0, 0)),
          pl.BlockSpec((N_META, H, H), lambda i: (0, 0, 0)),
          pl.BlockSpec((N_META, 1, H), lambda i: (0, 0, 0)),
      ],
      out_specs=[
          pl.BlockSpec((BN, H), lambda i: (i, 0)),
          pl.BlockSpec((BN, H), lambda i: (i, 0)),
      ],
      out_shape=[
          jax.ShapeDtypeStruct((N, H), jnp.float32),
          jax.ShapeDtypeStruct((N, H), jnp.float32),
      ],
  )(node_feats, node_types.reshape(N, 1).astype(jnp.int32),
    w_types, b_types.reshape(T, 1, H), affine_w,
    affine_b.reshape(N_META, 1, H))


def _make_tc_b(nc):
  def body(p_ref, s0_ref, s1_ref):
    s0 = p_ref[0, 0]
    s1 = p_ref[1, 0]
    for c in range(1, nc):
      s0 = s0 + p_ref[0, c]
      s1 = s1 + p_ref[1, c]
    s0_ref[...] = s0
    s1_ref[...] = s1

  def run(p):
    grid = (N // BN,)
    return pl.pallas_call(
        body,
        grid=grid,
        in_specs=[pl.BlockSpec((N_META, nc, BN, H), lambda i: (0, 0, i, 0))],
        out_specs=[
            pl.BlockSpec((BN, H), lambda i: (i, 0)),
            pl.BlockSpec((BN, H), lambda i: (i, 0)),
        ],
        out_shape=[
            jax.ShapeDtypeStruct((N, H), jnp.float32),
            jax.ShapeDtypeStruct((N, H), jnp.float32),
        ],
    )(p)

  return run


def _make_tc_c(nc):
  def body(p_ref, lns_ref, lnb_ref, w1_ref, b1_ref, w2_ref, b2_ref,
           cw_ref, cb_ref, out_ref):
    outs = []
    atts = []
    for m in range(N_META):
      s = p_ref[m, 0]
      for c in range(1, nc):
        s = s + p_ref[m, c]
      mu = jnp.mean(s, axis=-1, keepdims=True)
      var = jnp.mean((s - mu) ** 2, axis=-1, keepdims=True)
      y = (s - mu) * lax.rsqrt(var + 1e-5) * lns_ref[m] + lnb_ref[m]
      y = y * 0.5 * (1.0 + lax.erf(y * 0.7071067811865476))
      outs.append(y)
      z = jnp.tanh(jnp.dot(y, w1_ref[...],
                           preferred_element_type=jnp.float32) + b1_ref[...])
      a = jnp.sum(z * w2_ref[...], axis=-1, keepdims=True) + b2_ref[...]
      atts.append(a)
    mx = jnp.maximum(atts[0], atts[1])
    e0 = jnp.exp(atts[0] - mx)
    e1 = jnp.exp(atts[1] - mx)
    den = e0 + e1
    mix = outs[0] * (e0 / den) + outs[1] * (e1 / den)
    out_ref[...] = jnp.dot(mix, cw_ref[...],
                           preferred_element_type=jnp.float32) + cb_ref[...]

  def run(p, ln_scale, ln_bias, attn_w1, attn_b1, attn_w2, attn_b2,
          cls_w, cls_b):
    grid = (N // BN,)
    return pl.pallas_call(
        body,
        grid=grid,
        in_specs=[
            pl.BlockSpec((N_META, nc, BN, H), lambda i: (0, 0, i, 0)),
            pl.BlockSpec((N_META, 1, H), lambda i: (0, 0, 0)),
            pl.BlockSpec((N_META, 1, H), lambda i: (0, 0, 0)),
            pl.BlockSpec((H, ATT), lambda i: (0, 0)),
            pl.BlockSpec((1, ATT), lambda i: (0, 0)),
            pl.BlockSpec((1, ATT), lambda i: (0, 0)),
            pl.BlockSpec((1, 1), lambda i: (0, 0)),
            pl.BlockSpec((H, C), lambda i: (0, 0)),
            pl.BlockSpec((1, C), lambda i: (0, 0)),
        ],
        out_specs=pl.BlockSpec((BN, C), lambda i: (i, 0)),
        out_shape=jax.ShapeDtypeStruct((N, C), jnp.float32),
    )(p, ln_scale.reshape(N_META, 1, H), ln_bias.reshape(N_META, 1, H),
      attn_w1, attn_b1.reshape(1, ATT), attn_w2.reshape(1, ATT),
      attn_b2.reshape(1, 1), cls_w, cls_b.reshape(1, C))

  return run


def kernel(node_feats, node_types, edge_index, edge_vals, idxes_seq,
           idxes_res, W_types, b_types, affine_W, affine_b, ln_scale,
           ln_bias, attn_W1, attn_b1, attn_W2, attn_b2, cls_W, cls_b):
  info = plsc.get_sparse_core_info()
  nc, ns = info.num_cores, info.num_subcores

  def edges_of(a):
    ei = lax.dynamic_index_in_dim(edge_index, a, axis=0, keepdims=False)
    ev = lax.dynamic_index_in_dim(edge_vals, a, axis=0, keepdims=False)
    return ei[1], ei[0], ev

  a_seq = [[idxes_seq[m, i, 0] for i in range(2)] for m in range(N_META)]
  a_res = [idxes_res[m, 0, 0] for m in range(N_META)]

  x0, x1 = _tc_a(node_feats, node_types, W_types, b_types, affine_W,
                 affine_b)
  zeros = jnp.zeros((N, H), jnp.float32)

  t1 = [edges_of(a_seq[0][0]), edges_of(a_seq[1][0])]
  src1 = jnp.concatenate([t[0] for t in t1])
  dst1 = jnp.concatenate([t[1] for t in t1])
  val1 = jnp.concatenate([t[2] for t in t1])
  sc1 = _make_sc_spmm(nout=2, ntables=2, terms=[(0, 0), (1, 1)],
                      nc=nc, ns=ns)
  p1 = sc1(x0, x1, src1, dst1, val1, zeros)

  s1_0, s1_1 = _make_tc_b(nc)(p1)

  t2 = [edges_of(a_seq[0][1]), edges_of(a_res[0]),
        edges_of(a_seq[1][1]), edges_of(a_res[1])]
  src2 = jnp.concatenate([t[0] for t in t2])
  dst2 = jnp.concatenate([t[1] for t in t2])
  val2 = jnp.concatenate([t[2] for t in t2])
  sc2 = _make_sc_spmm(nout=2, ntables=4,
                      terms=[(0, 0), (0, 2), (1, 1), (1, 3)],
                      nc=nc, ns=ns)
  p2 = sc2(s1_0, s1_1, x0, x1, src2, dst2, val2, zeros)

  return _make_tc_c(nc)(p2, ln_scale, ln_bias, attn_W1, attn_b1,
                        attn_W2, attn_b2, cls_W, cls_b)


# speedup vs baseline: 7.4579x; 2.3704x over previous
import functools

import jax
import jax.numpy as jnp
from jax import lax
from jax.experimental import pallas as pl
from jax.experimental.pallas import tpu as pltpu
from jax.experimental.pallas import tpu_sc as plsc

N = 10000
E = 320000
IN_DIM = 128
H = 128
T = 4
C = 16
ATT = 64
N_META = 2

BN = 1000
K = 80


def _make_sc_spmm(nout, ntables, terms, nc, ns):
  nw = nc * ns
  we = E // nw
  cpb = 25
  sb_e = cpb * K
  nblk = we // sb_e
  rpw = (N // ns) & ~7
  rem = N - rpw * ns
  mesh = plsc.VectorSubcoreMesh(core_axis_name="c", subcore_axis_name="s")

  @functools.partial(
      pl.kernel,
      mesh=mesh,
      out_type=jax.ShapeDtypeStruct((nout, nc, N, H), jnp.float32),
      scratch_types=[
          pltpu.VMEM((sb_e,), jnp.int32),
          pltpu.VMEM((cpb, K), jnp.int32),
          pltpu.VMEM((sb_e,), jnp.float32),
          pltpu.VMEM((K, H), jnp.float32),
          pltpu.VMEM((K, H), jnp.float32),
          pltpu.VMEM_SHARED((N, H), jnp.float32),
          pltpu.SemaphoreType.DMA,
          pltpu.SemaphoreType.DMA,
      ],
  )
  def sc_fn(*refs):
    tables = refs[:ntables]
    src_h, dst_h, val_h, zeros_h, out_h = refs[ntables:ntables + 5]
    src_v, dst2d, val_v, rows0, rows1, acc_s, g0, g1 = refs[ntables + 5:]
    cid = lax.axis_index("c")
    sid = lax.axis_index("s")
    wid = sid * nc + cid

    def scale(buf, voff):
      def grp(i16, _):
        vv = val_v[pl.ds(voff + i16 * 16, 16)]
        for j in range(16):
          v = vv[j]
          i = i16 * 16 + j
          for g in range(H // 16):
            sl = pl.ds(g * 16, 16)
            buf[i, sl] = buf[i, sl] * v
        return 0

      lax.fori_loop(0, K // 16, grp, 0)

    for o in range(nout):
      pltpu.sync_copy(zeros_h.at[pl.ds(sid * rpw, rpw)],
                      acc_s.at[pl.ds(sid * rpw, rpw)])
      if rem:
        @pl.when(sid == ns - 1)
        def _():
          pltpu.sync_copy(zeros_h.at[pl.ds(N - rem, rem)],
                          acc_s.at[pl.ds(N - rem, rem)])
      plsc.subcore_barrier()
      for t, (oo, tbl) in enumerate(terms):
        if oo != o:
          continue
        table = tables[tbl]

        def block(sb, _, t=t, table=table):
          base = t * E + wid * we + sb * sb_e
          pltpu.sync_copy(src_h.at[pl.ds(base, sb_e)], src_v)
          pltpu.sync_copy(val_h.at[pl.ds(base, sb_e)], val_v)
          pltpu.sync_copy(dst_h.at[(t * nw + wid) * nblk + sb], dst2d)

          def gather(ci, buf, sem):
            pltpu.async_copy(table.at[src_v.at[pl.ds(ci * K, K)]], buf,
                             sem)

          def gwait(ci, buf, sem):
            pltpu.make_async_copy(table.at[src_v.at[pl.ds(ci * K, K)]],
                                  buf, sem).wait()

          def process(ci, buf):
            scale(buf, ci * K)
            pltpu.sync_copy(buf, acc_s.at[dst2d.at[ci]], add=True)

          gather(0, rows0, g0)

          def pair(i, _):
            ci0 = 2 * i
            ci1 = ci0 + 1
            gather(ci1, rows1, g1)
            gwait(ci0, rows0, g0)
            process(ci0, rows0)

            @pl.when(ci0 + 2 < cpb)
            def _():
              gather(ci0 + 2, rows0, g0)

            gwait(ci1, rows1, g1)
            process(ci1, rows1)
            return 0

          lax.fori_loop(0, cpb // 2, pair, 0)
          if cpb % 2:
            gwait(cpb - 1, rows0, g0)
            process(cpb - 1, rows0)
          return 0

        lax.fori_loop(0, nblk, block, 0)
      plsc.subcore_barrier()
      pltpu.sync_copy(acc_s.at[pl.ds(sid * rpw, rpw)],
                      out_h.at[o, cid, pl.ds(sid * rpw, rpw)])
      if rem:
        @pl.when(sid == ns - 1)
        def _():
          pltpu.sync_copy(acc_s.at[pl.ds(N - rem, rem)],
                          out_h.at[o, cid, pl.ds(N - rem, rem)])
      plsc.subcore_barrier()

  return sc_fn


def _tc_a_body(nf_ref, nt_ref, wt_ref, bt_ref, aw_ref, ab_ref,
               x0_ref, x1_ref):
  x = nf_ref[...]
  t = nt_ref[...]
  h = jnp.zeros((BN, H), jnp.float32)
  for tt in range(T):
    ht = jnp.dot(x, wt_ref[tt], preferred_element_type=jnp.float32)
    ht = ht + bt_ref[tt]
    h = jnp.where(t == tt, ht, h)
  x0_ref[...] = jnp.dot(h, aw_ref[0],
                        preferred_element_type=jnp.float32) + ab_ref[0]
  x1_ref[...] = jnp.dot(h, aw_ref[1],
                        preferred_element_type=jnp.float32) + ab_ref[1]


def _tc_a(node_feats, node_types, w_types, b_types, affine_w, affine_b):
  grid = (N // BN,)
  return pl.pallas_call(
      _tc_a_body,
      grid=grid,
      in_specs=[
          pl.BlockSpec((BN, IN_DIM), lambda i: (i, 0)),
          pl.BlockSpec((BN, 1), lambda i: (i, 0)),
          pl.BlockSpec((T, IN_DIM, H), lambda i: (0, 0, 0)),
          pl.BlockSpec((T, 1, H), lambda i: (0, 0, 0)),
          pl.BlockSpec((N_META, H, H), lambda i: (0, 0, 0)),
          pl.BlockSpec((N_META, 1, H), lambda i: (0, 0, 0)),
      ],
      out_specs=[
          pl.BlockSpec((BN, H), lambda i: (i, 0)),
          pl.BlockSpec((BN, H), lambda i: (i, 0)),
      ],
      out_shape=[
          jax.ShapeDtypeStruct((N, H), jnp.float32),
          jax.ShapeDtypeStruct((N, H), jnp.float32),
      ],
  )(node_feats, node_types.reshape(N, 1).astype(jnp.int32),
    w_types, b_types.reshape(T, 1, H), affine_w,
    affine_b.reshape(N_META, 1, H))


def _make_tc_b(nc):
  def body(p_ref, s0_ref, s1_ref):
    s0 = p_ref[0, 0]
    s1 = p_ref[1, 0]
    for c in range(1, nc):
      s0 = s0 + p_ref[0, c]
      s1 = s1 + p_ref[1, c]
    s0_ref[...] = s0
    s1_ref[...] = s1

  def run(p):
    grid = (N // BN,)
    return pl.pallas_call(
        body,
        grid=grid,
        in_specs=[pl.BlockSpec((N_META, nc, BN, H), lambda i: (0, 0, i, 0))],
        out_specs=[
            pl.BlockSpec((BN, H), lambda i: (i, 0)),
            pl.BlockSpec((BN, H), lambda i: (i, 0)),
        ],
        out_shape=[
            jax.ShapeDtypeStruct((N, H), jnp.float32),
            jax.ShapeDtypeStruct((N, H), jnp.float32),
        ],
    )(p)

  return run


def _make_tc_c(nc):
  def body(p_ref, lns_ref, lnb_ref, w1_ref, b1_ref, w2_ref, b2_ref,
           cw_ref, cb_ref, out_ref):
    outs = []
    atts = []
    for m in range(N_META):
      s = p_ref[m, 0]
      for c in range(1, nc):
        s = s + p_ref[m, c]
      mu = jnp.mean(s, axis=-1, keepdims=True)
      var = jnp.mean((s - mu) ** 2, axis=-1, keepdims=True)
      y = (s - mu) * lax.rsqrt(var + 1e-5) * lns_ref[m] + lnb_ref[m]
      y = y * 0.5 * (1.0 + lax.erf(y * 0.7071067811865476))
      outs.append(y)
      z = jnp.tanh(jnp.dot(y, w1_ref[...],
                           preferred_element_type=jnp.float32) + b1_ref[...])
      a = jnp.sum(z * w2_ref[...], axis=-1, keepdims=True) + b2_ref[...]
      atts.append(a)
    mx = jnp.maximum(atts[0], atts[1])
    e0 = jnp.exp(atts[0] - mx)
    e1 = jnp.exp(atts[1] - mx)
    den = e0 + e1
    mix = outs[0] * (e0 / den) + outs[1] * (e1 / den)
    out_ref[...] = jnp.dot(mix, cw_ref[...],
                           preferred_element_type=jnp.float32) + cb_ref[...]

  def run(p, ln_scale, ln_bias, attn_w1, attn_b1, attn_w2, attn_b2,
          cls_w, cls_b):
    grid = (N // BN,)
    return pl.pallas_call(
        body,
        grid=grid,
        in_specs=[
            pl.BlockSpec((N_META, nc, BN, H), lambda i: (0, 0, i, 0)),
            pl.BlockSpec((N_META, 1, H), lambda i: (0, 0, 0)),
            pl.BlockSpec((N_META, 1, H), lambda i: (0, 0, 0)),
            pl.BlockSpec((H, ATT), lambda i: (0, 0)),
            pl.BlockSpec((1, ATT), lambda i: (0, 0)),
            pl.BlockSpec((1, ATT), lambda i: (0, 0)),
            pl.BlockSpec((1, 1), lambda i: (0, 0)),
            pl.BlockSpec((H, C), lambda i: (0, 0)),
            pl.BlockSpec((1, C), lambda i: (0, 0)),
        ],
        out_specs=pl.BlockSpec((BN, C), lambda i: (i, 0)),
        out_shape=jax.ShapeDtypeStruct((N, C), jnp.float32),
    )(p, ln_scale.reshape(N_META, 1, H), ln_bias.reshape(N_META, 1, H),
      attn_w1, attn_b1.reshape(1, ATT), attn_w2.reshape(1, ATT),
      attn_b2.reshape(1, 1), cls_w, cls_b.reshape(1, C))

  return run


def kernel(node_feats, node_types, edge_index, edge_vals, idxes_seq,
           idxes_res, W_types, b_types, affine_W, affine_b, ln_scale,
           ln_bias, attn_W1, attn_b1, attn_W2, attn_b2, cls_W, cls_b):
  info = plsc.get_sparse_core_info()
  nc, ns = info.num_cores, info.num_subcores

  def edges_of(a):
    ei = lax.dynamic_index_in_dim(edge_index, a, axis=0, keepdims=False)
    ev = lax.dynamic_index_in_dim(edge_vals, a, axis=0, keepdims=False)
    return ei[1], ei[0], ev

  a_seq = [[idxes_seq[m, i, 0] for i in range(2)] for m in range(N_META)]
  a_res = [idxes_res[m, 0, 0] for m in range(N_META)]

  x0, x1 = _tc_a(node_feats, node_types, W_types, b_types, affine_W,
                 affine_b)
  zeros = jnp.zeros((N, H), jnp.float32)

  t1 = [edges_of(a_seq[0][0]), edges_of(a_seq[1][0])]
  src1 = jnp.concatenate([t[0] for t in t1])
  dst1 = jnp.concatenate([t[1] for t in t1])
  val1 = jnp.concatenate([t[2] for t in t1])
  sc1 = _make_sc_spmm(nout=2, ntables=2, terms=[(0, 0), (1, 1)],
                      nc=nc, ns=ns)
  p1 = sc1(x0, x1, src1, dst1.reshape(-1, 25, K), val1, zeros)

  s1_0, s1_1 = _make_tc_b(nc)(p1)

  t2 = [edges_of(a_seq[0][1]), edges_of(a_res[0]),
        edges_of(a_seq[1][1]), edges_of(a_res[1])]
  src2 = jnp.concatenate([t[0] for t in t2])
  dst2 = jnp.concatenate([t[1] for t in t2])
  val2 = jnp.concatenate([t[2] for t in t2])
  sc2 = _make_sc_spmm(nout=2, ntables=4,
                      terms=[(0, 0), (0, 2), (1, 1), (1, 3)],
                      nc=nc, ns=ns)
  p2 = sc2(s1_0, s1_1, x0, x1, src2, dst2.reshape(-1, 25, K), val2,
           zeros)

  return _make_tc_c(nc)(p2, ln_scale, ln_bias, attn_W1, attn_b1,
                        attn_W2, attn_b2, cls_W, cls_b)
